# Initial kernel scaffold; baseline (speedup 1.0000x reference)
import jax
import jax.numpy as jnp
from jax.experimental import pallas as pl


def kernel(positions, features, params):
    raise NotImplementedError("write your pallas kernel here")


# speedup vs baseline: 8.2940x; 8.2940x over previous
import jax
import jax.numpy as jnp
import numpy as np
from jax.experimental import pallas as pl
from jax.experimental.pallas import tpu as pltpu

BATCH = 64
N = 128
HID = 128
EMB_DIM = 32
N_FEAT = 5
N_BLOCKS = 3
N_INV_OUT = 64
ST = 64


def _silu(v):
    return v * jax.lax.logistic(v)


def _dot(a, b):
    return jax.lax.dot_general(a, b, (((1,), (0,)), ((), ())),
                               preferred_element_type=jnp.float32,
                               precision=jax.lax.Precision.HIGHEST)


def _bdot(a, b):
    return jax.lax.dot_general(a.astype(jnp.bfloat16), b.astype(jnp.bfloat16),
                               (((1,), (0,)), ((), ())),
                               preferred_element_type=jnp.float32)


def _b16(v):
    return v.astype(jnp.bfloat16).astype(jnp.float32)


def _egnn_body(x_ref, feat_ref, emb_ref, w_in_ref, b_in_ref,
               a1_ref, b1_ref, w1_ref, be1_ref, we2_ref, be2_ref, wx_ref,
               ah_ref, bh_ref, bh1_ref, wh2_ref, bh2_ref,
               w_out_ref, b_out_ref,
               vec_ref, sca_ref):
    f32 = jnp.float32
    x = x_ref[0]
    feat = feat_ref[0, 0]

    cols = jax.lax.broadcasted_iota(jnp.int32, (N, N_FEAT), 1)
    onehot = (feat[:, None] == cols).astype(f32)
    h = _dot(onehot, emb_ref[...])
    h = _bdot(h, w_in_ref[...]) + b_in_ref[...]

    nm1 = f32(N - 1)

    for i in range(N_BLOCKS):
        xn = jnp.sum(x * x, axis=1)
        gram = jax.lax.dot_general(x, x, (((1,), (1,)), ((), ())),
                                   preferred_element_type=f32,
                                   precision=jax.lax.Precision.HIGHEST)
        sq = jnp.maximum(xn[:, None] + xn[None, :] - 2.0 * gram, 0.0)

        hs_all = _bdot(h, a1_ref[i]) + be1_ref[i][None, :]
        hr_all = _bdot(h, b1_ref[i])
        w1b = _b16(w1_ref[i])
        we2 = we2_ref[i]
        be2 = be2_ref[i][None, :]
        wx = wx_ref[i]

        diag_m2 = _silu(_bdot(_silu(hs_all + hr_all), we2) + be2)

        colsum = jnp.zeros((N, HID), f32)
        cxa = jnp.zeros((N, 4), f32)
        ones_col = jnp.ones((ST, 1), f32)
        for s0 in range(0, N, ST):
            hs = hs_all[s0:s0 + ST]
            sq_c = sq[s0:s0 + ST]
            sq_b = _b16(sq_c)
            m1 = hs[:, None, :] + hr_all[None, :, :] + sq_b[:, :, None] * w1b[None, None, :]
            m1 = _silu(m1).reshape(ST * N, HID)
            m2 = _silu(_bdot(m1, we2) + be2)
            gate = _bdot(m2, wx)
            colsum = colsum + jnp.sum(m2.reshape(ST, N, HID), axis=0)
            coef = gate.reshape(ST, N) / (jnp.sqrt(sq_c) + 1.0)
            xs_aug = jnp.concatenate([x[s0:s0 + ST], ones_col], axis=1)
            cxa = cxa + jax.lax.dot_general(coef, xs_aug, (((0,), (0,)), ((), ())),
                                            preferred_element_type=f32,
                                            precision=jax.lax.Precision.HIGHEST)

        agg_m = colsum - diag_m2
        cxs, csum = cxa[:, :3], cxa[:, 3:4]
        x = x + (cxs - x * csum) / nm1

        dh = _silu(_bdot(h, ah_ref[i]) + _bdot(agg_m, bh_ref[i]) + bh1_ref[i][None, :])
        h = h + _bdot(dh, wh2_ref[i]) + bh2_ref[i][None, :]

    hmax = jnp.max(h, axis=1, keepdims=True)
    e = jnp.exp(h - hmax)
    p = e / jnp.sum(e, axis=1, keepdims=True)
    sca = _bdot(p, w_out_ref[...]) + b_out_ref[...]

    vec_ref[0] = x
    sca_ref[0] = sca


def kernel(positions, features, params):
    x = positions.reshape(BATCH, N, 3).astype(jnp.float32)
    feat = features.reshape(BATCH, 1, N).astype(jnp.int32)

    blocks = params['blocks']
    stack = lambda f: jnp.stack([f(b) for b in blocks])
    a1 = stack(lambda b: b['We1'][:HID])
    b1 = stack(lambda b: b['We1'][HID:2 * HID])
    w1 = stack(lambda b: b['We1'][2 * HID])
    be1 = stack(lambda b: b['be1'])
    we2 = stack(lambda b: b['We2'])
    be2 = stack(lambda b: b['be2'])
    wx = stack(lambda b: b['Wx'])
    ah = stack(lambda b: b['Wh1'][:HID])
    bh = stack(lambda b: b['Wh1'][HID:])
    bh1 = stack(lambda b: b['bh1'])
    wh2 = stack(lambda b: b['Wh2'])
    bh2 = stack(lambda b: b['bh2'])

    emb = params['embedding'].astype(jnp.float32)
    w_in = params['W_in']
    b_in = params['b_in'].reshape(1, HID)
    w_out = params['W_out']
    b_out = params['b_out'].reshape(1, N_INV_OUT)

    const = lambda *shape: pl.BlockSpec(shape, lambda b: (0,) * len(shape))
    grid_spec = pl.GridSpec(
        grid=(BATCH,),
        in_specs=[
            pl.BlockSpec((1, N, 3), lambda b: (b, 0, 0)),
            pl.BlockSpec((1, 1, N), lambda b: (b, 0, 0)),
            const(N_FEAT, EMB_DIM),
            const(EMB_DIM, HID),
            const(1, HID),
            const(N_BLOCKS, HID, HID),
            const(N_BLOCKS, HID, HID),
            const(N_BLOCKS, HID),
            const(N_BLOCKS, HID),
            const(N_BLOCKS, HID, HID),
            const(N_BLOCKS, HID),
            const(N_BLOCKS, HID, 1),
            const(N_BLOCKS, HID, HID),
            const(N_BLOCKS, HID, HID),
            const(N_BLOCKS, HID),
            const(N_BLOCKS, HID, HID),
            const(N_BLOCKS, HID),
            const(HID, N_INV_OUT),
            const(1, N_INV_OUT),
        ],
        out_specs=[
            pl.BlockSpec((1, N, 3), lambda b: (b, 0, 0)),
            pl.BlockSpec((1, N, N_INV_OUT), lambda b: (b, 0, 0)),
        ],
    )
    vec, sca = pl.pallas_call(
        _egnn_body,
        grid_spec=grid_spec,
        out_shape=[
            jax.ShapeDtypeStruct((BATCH, N, 3), jnp.float32),
            jax.ShapeDtypeStruct((BATCH, N, N_INV_OUT), jnp.float32),
        ],
    )(x, feat, emb, w_in, b_in, a1, b1, w1, be1, we2, be2, wx,
      ah, bh, bh1, wh2, bh2, w_out, b_out)
    return vec.reshape(BATCH, N, 1, 3), sca
